# baseline (speedup 1.0000x reference)
import jax
import jax.numpy as jnp
from jax import lax
from jax.experimental import pallas as pl
from jax.experimental.pallas import tpu as pltpu

NUM_OUTPUTS = 1000
ROWS = 4096
COLS = 26
BLOCK_R = 32
NBUF = 8
NBLK = ROWS // BLOCK_R
N_OUTER = NBLK // NBUF


def _body(idx_ref, out_ref, *scratch):
    bufs = scratch[:NBUF]
    sems = scratch[NBUF]

    def outer(o, carry):
        for b in range(NBUF):
            i = o * NBUF + b
            base = pl.multiple_of(i * BLOCK_R, BLOCK_R)

            @pl.when(o > 0)
            def _wait():
                pltpu.make_async_copy(
                    bufs[b], out_ref.at[pl.ds(base, BLOCK_R)], sems.at[b]
                ).wait()

            idx_blk = idx_ref[pl.ds(base, BLOCK_R), :]
            iota = lax.broadcasted_iota(
                jnp.int32, (BLOCK_R, COLS, NUM_OUTPUTS), 2
            )
            bufs[b][...] = (iota == idx_blk[:, :, None]).astype(jnp.int32)
            pltpu.make_async_copy(
                bufs[b], out_ref.at[pl.ds(base, BLOCK_R)], sems.at[b]
            ).start()
        return carry

    lax.fori_loop(0, N_OUTER, outer, 0)
    for b in range(NBUF):
        pltpu.make_async_copy(
            bufs[b], out_ref.at[pl.ds(b * BLOCK_R, BLOCK_R)], sems.at[b]
        ).wait()


def kernel(inputs):
    return pl.pallas_call(
        _body,
        in_specs=[pl.BlockSpec(memory_space=pltpu.MemorySpace.VMEM)],
        out_specs=pl.BlockSpec(memory_space=pltpu.MemorySpace.HBM),
        out_shape=jax.ShapeDtypeStruct((ROWS, COLS, NUM_OUTPUTS), jnp.int32),
        scratch_shapes=(
            [pltpu.VMEM((BLOCK_R, COLS, NUM_OUTPUTS), jnp.int32)] * NBUF
            + [pltpu.SemaphoreType.DMA((NBUF,))]
        ),
    )(inputs)
